# Initial kernel scaffold; baseline (speedup 1.0000x reference)
import functools

import jax
import jax.numpy as jnp
from jax import lax
from jax.experimental import pallas as pl
from jax.experimental.pallas import tpu as pltpu
from jax.experimental.pallas import tpu_sc as plsc

_N_FIELDS = 26
_VOCAB = 100000
_EMB_DIM = 16
_SEQ_LEN = 20
_BATCH = 4096

_N_ROWS = _SEQ_LEN * _BATCH * _N_FIELDS
_NUM_WORKERS = 32
_ROWS_PER_WORKER = _N_ROWS // _NUM_WORKERS
_CHUNK = 2080
_CHUNKS_PER_WORKER = _ROWS_PER_WORKER // _CHUNK
_LANES = 16


@functools.partial(
    pl.kernel,
    mesh=plsc.VectorSubcoreMesh(core_axis_name="c", subcore_axis_name="s"),
    out_type=jax.ShapeDtypeStruct((_N_ROWS, _EMB_DIM), jnp.float32),
    scratch_types=[
        pltpu.VMEM((_CHUNK,), jnp.int32),
        pltpu.VMEM((_CHUNK,), jnp.int32),
        pltpu.VMEM((_CHUNK, _EMB_DIM), jnp.float32),
        pltpu.SemaphoreType.DMA,
    ],
)
def _gather_kernel(tables_hbm, idx_hbm, off_hbm, out_hbm,
                   off_v, idx_v, rows_v, sem):
    wid = lax.axis_index("s") * 2 + lax.axis_index("c")
    base0 = wid * _ROWS_PER_WORKER

    pltpu.sync_copy(off_hbm, off_v)

    def chunk_body(c, carry):
        base = base0 + c * _CHUNK
        pltpu.sync_copy(idx_hbm.at[pl.ds(base, _CHUNK)], idx_v)

        def add_body(i, carry2):
            sl = pl.ds(pl.multiple_of(i * _LANES, _LANES), _LANES)
            idx_v[sl] = idx_v[sl] + off_v[sl]
            return carry2

        lax.fori_loop(0, _CHUNK // _LANES, add_body, 0)

        pltpu.async_copy(tables_hbm.at[idx_v], rows_v, sem).wait()
        pltpu.sync_copy(rows_v, out_hbm.at[pl.ds(base, _CHUNK)])
        return carry

    lax.fori_loop(0, _CHUNKS_PER_WORKER, chunk_body, 0)


def kernel(cat_tensor, tables):
    tables_flat = tables.reshape(_N_FIELDS * _VOCAB, _EMB_DIM)
    idx_flat = cat_tensor.reshape(_N_ROWS)
    off = (jnp.arange(_CHUNK, dtype=jnp.int32) % _N_FIELDS) * _VOCAB
    out_flat = _gather_kernel(tables_flat, idx_flat, off)
    return out_flat.reshape(_SEQ_LEN, _BATCH, _N_FIELDS * _EMB_DIM)


# speedup vs baseline: 1.4347x; 1.4347x over previous
import functools

import jax
import jax.numpy as jnp
from jax import lax
from jax.experimental import pallas as pl
from jax.experimental.pallas import tpu as pltpu
from jax.experimental.pallas import tpu_sc as plsc

_N_FIELDS = 26
_VOCAB = 100000
_EMB_DIM = 16
_SEQ_LEN = 20
_BATCH = 4096

_N_ROWS = _SEQ_LEN * _BATCH * _N_FIELDS
_NUM_WORKERS = 32
_ROWS_PER_WORKER = _N_ROWS // _NUM_WORKERS
_CHUNK = 2080
_CHUNKS_PER_WORKER = _ROWS_PER_WORKER // _CHUNK
_LANES = 16


@functools.partial(
    pl.kernel,
    mesh=plsc.VectorSubcoreMesh(core_axis_name="c", subcore_axis_name="s"),
    out_type=jax.ShapeDtypeStruct((_N_ROWS, _EMB_DIM), jnp.float32),
    scratch_types=[
        pltpu.VMEM((_CHUNK,), jnp.int32),
        pltpu.VMEM((_CHUNK,), jnp.int32),
        pltpu.VMEM((_CHUNK, _EMB_DIM), jnp.float32),
        pltpu.SemaphoreType.DMA,
    ],
    compiler_params=pltpu.CompilerParams(use_tc_tiling_on_sc=False),
)
def _gather_kernel(tables_hbm, idx_hbm, off_hbm, out_hbm,
                   off_v, idx_v, rows_v, sem):
    wid = lax.axis_index("s") * 2 + lax.axis_index("c")
    base0 = wid * _ROWS_PER_WORKER

    pltpu.sync_copy(off_hbm, off_v)

    def chunk_body(c, carry):
        base = base0 + c * _CHUNK
        pltpu.sync_copy(idx_hbm.at[pl.ds(base, _CHUNK)], idx_v)

        def add_body(i, carry2):
            sl = pl.ds(pl.multiple_of(i * _LANES, _LANES), _LANES)
            idx_v[sl] = idx_v[sl] + off_v[sl]
            return carry2

        lax.fori_loop(0, _CHUNK // _LANES, add_body, 0)

        pltpu.async_copy(tables_hbm.at[idx_v], rows_v, sem).wait()
        pltpu.sync_copy(rows_v, out_hbm.at[pl.ds(base, _CHUNK)])
        return carry

    lax.fori_loop(0, _CHUNKS_PER_WORKER, chunk_body, 0)


def kernel(cat_tensor, tables):
    tables_flat = tables.reshape(_N_FIELDS * _VOCAB, _EMB_DIM)
    idx_flat = cat_tensor.reshape(_N_ROWS)
    off = (jnp.arange(_CHUNK, dtype=jnp.int32) % _N_FIELDS) * _VOCAB
    out_flat = _gather_kernel(tables_flat, idx_flat, off)
    return out_flat.reshape(_SEQ_LEN, _BATCH, _N_FIELDS * _EMB_DIM)


# speedup vs baseline: 1.4611x; 1.0185x over previous
import functools

import jax
import jax.numpy as jnp
from jax import lax
from jax.experimental import pallas as pl
from jax.experimental.pallas import tpu as pltpu
from jax.experimental.pallas import tpu_sc as plsc

_N_FIELDS = 26
_VOCAB = 100000
_EMB_DIM = 16
_SEQ_LEN = 20
_BATCH = 4096

_N_ROWS = _SEQ_LEN * _BATCH * _N_FIELDS
_NUM_WORKERS = 32
_ROWS_PER_WORKER = _N_ROWS // _NUM_WORKERS
_CHUNK = 2080
_CHUNKS_PER_WORKER = _ROWS_PER_WORKER // _CHUNK
_LANES = 16


@functools.partial(
    pl.kernel,
    mesh=plsc.VectorSubcoreMesh(core_axis_name="c", subcore_axis_name="s"),
    out_type=jax.ShapeDtypeStruct((_N_ROWS, _EMB_DIM), jnp.float32),
    scratch_types=[
        pltpu.VMEM((_CHUNK,), jnp.int32),
        pltpu.VMEM((_CHUNK,), jnp.int32),
        pltpu.VMEM((_CHUNK, _EMB_DIM), jnp.float32),
        pltpu.VMEM((_CHUNK, _EMB_DIM), jnp.float32),
        pltpu.SemaphoreType.DMA,
        pltpu.SemaphoreType.DMA,
        pltpu.SemaphoreType.DMA,
    ],
    compiler_params=pltpu.CompilerParams(use_tc_tiling_on_sc=False),
)
def _gather_kernel(tables_hbm, idx_hbm, off_hbm, out_hbm,
                   off_v, idx_v, rows_a, rows_b, gsem, wsem_a, wsem_b):
    wid = lax.axis_index("s") * 2 + lax.axis_index("c")
    base0 = wid * _ROWS_PER_WORKER
    row_bufs = (rows_a, rows_b)
    wsems = (wsem_a, wsem_b)

    pltpu.sync_copy(off_hbm, off_v)

    w_desc = [None, None]
    for c in range(_CHUNKS_PER_WORKER):
        b = c % 2
        base = base0 + c * _CHUNK
        pltpu.sync_copy(idx_hbm.at[pl.ds(base, _CHUNK)], idx_v)

        def add_body(i, carry):
            sl = pl.ds(pl.multiple_of(i * _LANES, _LANES), _LANES)
            idx_v[sl] = idx_v[sl] + off_v[sl]
            return carry

        lax.fori_loop(0, _CHUNK // _LANES, add_body, 0)

        if w_desc[b] is not None:
            w_desc[b].wait()
        pltpu.async_copy(tables_hbm.at[idx_v], row_bufs[b], gsem).wait()
        w_desc[b] = pltpu.async_copy(row_bufs[b],
                                     out_hbm.at[pl.ds(base, _CHUNK)], wsems[b])

    w_desc[0].wait()
    w_desc[1].wait()


def kernel(cat_tensor, tables):
    tables_flat = tables.reshape(_N_FIELDS * _VOCAB, _EMB_DIM)
    idx_flat = cat_tensor.reshape(_N_ROWS)
    off = (jnp.arange(_CHUNK, dtype=jnp.int32) % _N_FIELDS) * _VOCAB
    out_flat = _gather_kernel(tables_flat, idx_flat, off)
    return out_flat.reshape(_SEQ_LEN, _BATCH, _N_FIELDS * _EMB_DIM)


# speedup vs baseline: 1.4786x; 1.0119x over previous
import functools

import jax
import jax.numpy as jnp
from jax import lax
from jax.experimental import pallas as pl
from jax.experimental.pallas import tpu as pltpu
from jax.experimental.pallas import tpu_sc as plsc

_N_FIELDS = 26
_VOCAB = 100000
_EMB_DIM = 16
_SEQ_LEN = 20
_BATCH = 4096

_N_ROWS = _SEQ_LEN * _BATCH * _N_FIELDS
_NUM_WORKERS = 32
_ROWS_PER_WORKER = _N_ROWS // _NUM_WORKERS
_CHUNK = 2080
_CHUNKS_PER_WORKER = _ROWS_PER_WORKER // _CHUNK
_LANES = 16


@functools.partial(
    pl.kernel,
    mesh=plsc.VectorSubcoreMesh(core_axis_name="c", subcore_axis_name="s"),
    out_type=jax.ShapeDtypeStruct((_N_ROWS, _EMB_DIM), jnp.float32),
    scratch_types=[
        pltpu.VMEM((_CHUNK,), jnp.int32),
        pltpu.VMEM((_CHUNK,), jnp.int32),
        pltpu.VMEM((_CHUNK,), jnp.int32),
        pltpu.VMEM((_CHUNK, _EMB_DIM), jnp.float32),
        pltpu.VMEM((_CHUNK, _EMB_DIM), jnp.float32),
        pltpu.SemaphoreType.DMA,
        pltpu.SemaphoreType.DMA,
        pltpu.SemaphoreType.DMA,
        pltpu.SemaphoreType.DMA,
    ],
    compiler_params=pltpu.CompilerParams(use_tc_tiling_on_sc=False),
)
def _gather_kernel(tables_hbm, idx_hbm, off_hbm, out_hbm,
                   off_v, idx_a, idx_b, rows_a, rows_b,
                   gsem_a, gsem_b, wsem_a, wsem_b):
    wid = lax.axis_index("s") * 2 + lax.axis_index("c")
    base0 = wid * _ROWS_PER_WORKER
    idx_bufs = (idx_a, idx_b)
    row_bufs = (rows_a, rows_b)
    gsems = (gsem_a, gsem_b)
    wsems = (wsem_a, wsem_b)

    pltpu.sync_copy(off_hbm, off_v)

    def stage(c, idx_v):
        pltpu.sync_copy(idx_hbm.at[pl.ds(base0 + c * _CHUNK, _CHUNK)], idx_v)

        def add_body(i, carry):
            sl = pl.ds(pl.multiple_of(i * _LANES, _LANES), _LANES)
            idx_v[sl] = idx_v[sl] + off_v[sl]
            return carry

        lax.fori_loop(0, _CHUNK // _LANES, add_body, 0)

    stage(0, idx_bufs[0])
    g_desc = [pltpu.async_copy(tables_hbm.at[idx_bufs[0]], row_bufs[0],
                               gsems[0]), None]
    w_desc = [None, None]

    for c in range(_CHUNKS_PER_WORKER):
        b = c % 2
        nb = 1 - b
        if c + 1 < _CHUNKS_PER_WORKER:
            stage(c + 1, idx_bufs[nb])
        g_desc[b].wait()
        if c + 1 < _CHUNKS_PER_WORKER:
            if w_desc[nb] is not None:
                w_desc[nb].wait()
            g_desc[nb] = pltpu.async_copy(tables_hbm.at[idx_bufs[nb]],
                                          row_bufs[nb], gsems[nb])
        w_desc[b] = pltpu.async_copy(
            row_bufs[b], out_hbm.at[pl.ds(base0 + c * _CHUNK, _CHUNK)],
            wsems[b])

    w_desc[0].wait()
    w_desc[1].wait()


def kernel(cat_tensor, tables):
    tables_flat = tables.reshape(_N_FIELDS * _VOCAB, _EMB_DIM)
    idx_flat = cat_tensor.reshape(_N_ROWS)
    off = (jnp.arange(_CHUNK, dtype=jnp.int32) % _N_FIELDS) * _VOCAB
    out_flat = _gather_kernel(tables_flat, idx_flat, off)
    return out_flat.reshape(_SEQ_LEN, _BATCH, _N_FIELDS * _EMB_DIM)


# speedup vs baseline: 1.4969x; 1.0124x over previous
import functools

import jax
import jax.numpy as jnp
from jax import lax
from jax.experimental import pallas as pl
from jax.experimental.pallas import tpu as pltpu
from jax.experimental.pallas import tpu_sc as plsc

_N_FIELDS = 26
_VOCAB = 100000
_EMB_DIM = 16
_SEQ_LEN = 20
_BATCH = 4096

_N_ROWS = _SEQ_LEN * _BATCH * _N_FIELDS
_NUM_WORKERS = 32
_ROWS_PER_WORKER = _N_ROWS // _NUM_WORKERS
_CHUNK = 2080
_CHUNKS_PER_WORKER = _ROWS_PER_WORKER // _CHUNK
_LANES = 16
_NBUF = 3


@functools.partial(
    pl.kernel,
    mesh=plsc.VectorSubcoreMesh(core_axis_name="c", subcore_axis_name="s"),
    out_type=jax.ShapeDtypeStruct((_N_ROWS, _EMB_DIM), jnp.float32),
    scratch_types=[
        pltpu.VMEM((_CHUNK,), jnp.int32),
        pltpu.VMEM((_CHUNK,), jnp.int32),
        pltpu.VMEM((_CHUNK,), jnp.int32),
        pltpu.VMEM((_CHUNK,), jnp.int32),
        pltpu.VMEM((_CHUNK, _EMB_DIM), jnp.float32),
        pltpu.VMEM((_CHUNK, _EMB_DIM), jnp.float32),
        pltpu.VMEM((_CHUNK, _EMB_DIM), jnp.float32),
        pltpu.SemaphoreType.DMA,
        pltpu.SemaphoreType.DMA,
        pltpu.SemaphoreType.DMA,
        pltpu.SemaphoreType.DMA,
        pltpu.SemaphoreType.DMA,
        pltpu.SemaphoreType.DMA,
    ],
    compiler_params=pltpu.CompilerParams(use_tc_tiling_on_sc=False),
)
def _gather_kernel(tables_hbm, idx_hbm, off_hbm, out_hbm,
                   off_v, idx0, idx1, idx2, rows0, rows1, rows2,
                   gsem0, gsem1, gsem2, wsem0, wsem1, wsem2):
    wid = lax.axis_index("s") * 2 + lax.axis_index("c")
    base0 = wid * _ROWS_PER_WORKER
    idx_bufs = (idx0, idx1, idx2)
    row_bufs = (rows0, rows1, rows2)
    gsems = (gsem0, gsem1, gsem2)
    wsems = (wsem0, wsem1, wsem2)

    pltpu.sync_copy(off_hbm, off_v)

    def stage(c, idx_v):
        pltpu.sync_copy(idx_hbm.at[pl.ds(base0 + c * _CHUNK, _CHUNK)], idx_v)

        def add_body(i, carry):
            sl = pl.ds(pl.multiple_of(i * _LANES, _LANES), _LANES)
            idx_v[sl] = idx_v[sl] + off_v[sl]
            return carry

        lax.fori_loop(0, _CHUNK // _LANES, add_body, 0)

    def gather(k):
        return pltpu.async_copy(tables_hbm.at[idx_bufs[k]], row_bufs[k],
                                gsems[k])

    g_desc = [None] * _NBUF
    w_desc = [None] * _NBUF

    stage(0, idx_bufs[0])
    g_desc[0] = gather(0)
    stage(1, idx_bufs[1])
    g_desc[1] = gather(1)

    for c in range(_CHUNKS_PER_WORKER):
        k = c % _NBUF
        g_desc[k].wait()
        w_desc[k] = pltpu.async_copy(
            row_bufs[k], out_hbm.at[pl.ds(base0 + c * _CHUNK, _CHUNK)],
            wsems[k])
        if c + 2 < _CHUNKS_PER_WORKER:
            k2 = (c + 2) % _NBUF
            stage(c + 2, idx_bufs[k2])
            if w_desc[k2] is not None:
                w_desc[k2].wait()
            g_desc[k2] = gather(k2)

    for k in range(_NBUF):
        if w_desc[k] is not None:
            w_desc[k].wait()


def kernel(cat_tensor, tables):
    tables_flat = tables.reshape(_N_FIELDS * _VOCAB, _EMB_DIM)
    idx_flat = cat_tensor.reshape(_N_ROWS)
    off = (jnp.arange(_CHUNK, dtype=jnp.int32) % _N_FIELDS) * _VOCAB
    out_flat = _gather_kernel(tables_flat, idx_flat, off)
    return out_flat.reshape(_SEQ_LEN, _BATCH, _N_FIELDS * _EMB_DIM)


# speedup vs baseline: 1.5005x; 1.0024x over previous
import functools

import jax
import jax.numpy as jnp
from jax import lax
from jax.experimental import pallas as pl
from jax.experimental.pallas import tpu as pltpu
from jax.experimental.pallas import tpu_sc as plsc

_N_FIELDS = 26
_VOCAB = 100000
_EMB_DIM = 16
_SEQ_LEN = 20
_BATCH = 4096

_N_ROWS = _SEQ_LEN * _BATCH * _N_FIELDS
_NUM_WORKERS = 32
_ROWS_PER_WORKER = _N_ROWS // _NUM_WORKERS
_CHUNK = 1040
_CHUNKS_PER_WORKER = _ROWS_PER_WORKER // _CHUNK
_LANES = 16
_NBUF = 5
_INFLIGHT = _NBUF - 1


@functools.partial(
    pl.kernel,
    mesh=plsc.VectorSubcoreMesh(core_axis_name="c", subcore_axis_name="s"),
    out_type=jax.ShapeDtypeStruct((_N_ROWS, _EMB_DIM), jnp.float32),
    scratch_types=(
        [pltpu.VMEM((_CHUNK,), jnp.int32)]
        + [pltpu.VMEM((_CHUNK,), jnp.int32) for _ in range(_NBUF)]
        + [pltpu.VMEM((_CHUNK, _EMB_DIM), jnp.float32)
           for _ in range(_NBUF)]
        + [pltpu.SemaphoreType.DMA for _ in range(2 * _NBUF)]
    ),
    compiler_params=pltpu.CompilerParams(use_tc_tiling_on_sc=False),
)
def _gather_kernel(tables_hbm, idx_hbm, off_hbm, out_hbm, off_v, *scratch):
    idx_bufs = scratch[:_NBUF]
    row_bufs = scratch[_NBUF:2 * _NBUF]
    gsems = scratch[2 * _NBUF:3 * _NBUF]
    wsems = scratch[3 * _NBUF:4 * _NBUF]
    wid = lax.axis_index("s") * 2 + lax.axis_index("c")
    base0 = wid * _ROWS_PER_WORKER

    pltpu.sync_copy(off_hbm, off_v)

    def stage(c, idx_v):
        pltpu.sync_copy(idx_hbm.at[pl.ds(base0 + c * _CHUNK, _CHUNK)], idx_v)

        def add_body(i, carry):
            sl = pl.ds(pl.multiple_of(i * _LANES, _LANES), _LANES)
            idx_v[sl] = idx_v[sl] + off_v[sl]
            return carry

        lax.fori_loop(0, _CHUNK // _LANES, add_body, 0)

    def gather(k):
        return pltpu.async_copy(tables_hbm.at[idx_bufs[k]], row_bufs[k],
                                gsems[k])

    g_desc = [None] * _NBUF
    w_desc = [None] * _NBUF

    for j in range(_INFLIGHT):
        stage(j, idx_bufs[j])
        g_desc[j] = gather(j)

    for c in range(_CHUNKS_PER_WORKER):
        k = c % _NBUF
        g_desc[k].wait()
        w_desc[k] = pltpu.async_copy(
            row_bufs[k], out_hbm.at[pl.ds(base0 + c * _CHUNK, _CHUNK)],
            wsems[k])
        n = c + _INFLIGHT
        if n < _CHUNKS_PER_WORKER:
            k2 = n % _NBUF
            stage(n, idx_bufs[k2])
            if w_desc[k2] is not None:
                w_desc[k2].wait()
            g_desc[k2] = gather(k2)

    for k in range(_NBUF):
        if w_desc[k] is not None:
            w_desc[k].wait()


def kernel(cat_tensor, tables):
    tables_flat = tables.reshape(_N_FIELDS * _VOCAB, _EMB_DIM)
    idx_flat = cat_tensor.reshape(_N_ROWS)
    off = (jnp.arange(_CHUNK, dtype=jnp.int32) % _N_FIELDS) * _VOCAB
    out_flat = _gather_kernel(tables_flat, idx_flat, off)
    return out_flat.reshape(_SEQ_LEN, _BATCH, _N_FIELDS * _EMB_DIM)
